# baseline (speedup 1.0000x reference)
import jax
import jax.numpy as jnp
from jax.experimental import pallas as pl
from jax.experimental.pallas import tpu as pltpu

_WIDE = (
    (25600, 400),
    (19200, 400),
    (102400, 400),
    (236800, 400),
)
_CHUNK_ROWS = 3200
_NS = 8
_LAG = 4

_VSHAPE = (1250, 32, 65)
_VCHUNK = 125
_VN = _VSHAPE[0] // _VCHUNK

_CHUNKS = [(t, r0) for t, (rows, _) in enumerate(_WIDE)
           for r0 in range(0, rows, _CHUNK_ROWS)]
_VSTEPS = {8 + 10 * k: k for k in range(_VN)}


def _stream_body(s0, s1, s2, s3, vx, o0, o1, o2, o3, ov,
                 ring, vring, rsem, wsem, vrsem, vwsem):
    ins = (s0, s1, s2, s3)
    outs = (o0, o1, o2, o3)
    n = len(_CHUNKS)

    def rd(i):
        t, r0 = _CHUNKS[i]
        return pltpu.make_async_copy(
            ins[t].at[pl.ds(r0, _CHUNK_ROWS)], ring.at[i % _NS], rsem.at[i % _NS])

    def wr(i):
        t, r0 = _CHUNKS[i]
        return pltpu.make_async_copy(
            ring.at[i % _NS], outs[t].at[pl.ds(r0, _CHUNK_ROWS)], wsem.at[i % _NS])

    def vrd(k):
        return pltpu.make_async_copy(
            vx.at[pl.ds(k * _VCHUNK, _VCHUNK)], vring.at[k % 2], vrsem.at[k % 2])

    def vwr(k):
        return pltpu.make_async_copy(
            vring.at[k % 2], ov.at[pl.ds(k * _VCHUNK, _VCHUNK)], vwsem.at[k % 2])

    vrd(0).start()
    vrd(1).start()
    for i in range(n + _LAG):
        if i < n:
            if i >= _NS:
                wr(i - _NS).wait()
            rd(i).start()
        j = i - _LAG
        if 0 <= j < n:
            rd(j).wait()
            wr(j).start()
        k = _VSTEPS.get(i)
        if k is not None:
            if k >= 2:
                vwr(k - 2).wait()
            vrd(k).wait()
            vwr(k).start()
            if k + 2 < _VN:
                vrd(k + 2).start()
    for j in range(n - _NS, n):
        wr(j).wait()
    vwr(_VN - 2).wait()
    vwr(_VN - 1).wait()


def kernel(semantic, distance, vertex, embedding, direction):
    wide = [semantic.reshape(_WIDE[0]), distance.reshape(_WIDE[1]),
            embedding.reshape(_WIDE[2]), direction.reshape(_WIDE[3])]
    vx = vertex.transpose(2, 3, 0, 1).reshape(_VSHAPE)
    outs = pl.pallas_call(
        _stream_body,
        in_specs=[pl.BlockSpec(memory_space=pl.ANY)] * 5,
        out_specs=[pl.BlockSpec(memory_space=pl.ANY)] * 5,
        out_shape=[jax.ShapeDtypeStruct(f.shape, f.dtype)
                   for f in (*wide, vx)],
        scratch_shapes=[
            pltpu.VMEM((_NS, _CHUNK_ROWS, 400), jnp.float32),
            pltpu.VMEM((2, _VCHUNK) + _VSHAPE[1:], jnp.float32),
            pltpu.SemaphoreType.DMA((_NS,)),
            pltpu.SemaphoreType.DMA((_NS,)),
            pltpu.SemaphoreType.DMA((2,)),
            pltpu.SemaphoreType.DMA((2,)),
        ],
        compiler_params=pltpu.CompilerParams(vmem_limit_bytes=60 * 1024 * 1024),
    )(*wide, vx)
    ver = outs[4].reshape(25, 50, 32, 65).transpose(2, 3, 0, 1)
    return (outs[0].reshape(semantic.shape), outs[1].reshape(distance.shape),
            ver, outs[2].reshape(embedding.shape),
            outs[3].reshape(direction.shape))


# speedup vs baseline: 1.0013x; 1.0013x over previous
import jax
import jax.numpy as jnp
from jax.experimental import pallas as pl
from jax.experimental.pallas import tpu as pltpu

_WIDE = (
    (25600, 400),
    (19200, 400),
    (102400, 400),
    (236800, 400),
)
_CHUNK_ROWS = 3200
_NS = 8
_LAG = 4

_VSHAPE = (1250, 32, 65)
_VCHUNK = 125
_VN = _VSHAPE[0] // _VCHUNK

_CHUNKS = sorted(
    ((t, r0) for t, (rows, _) in enumerate(_WIDE)
     for r0 in range(0, rows, _CHUNK_ROWS)),
    key=lambda c: (c[1] / _WIDE[c[0]][0], c[0]))
_VSTEPS = {8 + 10 * k: k for k in range(_VN)}


def _stream_body(s0, s1, s2, s3, vx, o0, o1, o2, o3, ov,
                 ring, vring, rsem, wsem, vrsem, vwsem):
    ins = (s0, s1, s2, s3)
    outs = (o0, o1, o2, o3)
    n = len(_CHUNKS)

    def rd(i):
        t, r0 = _CHUNKS[i]
        return pltpu.make_async_copy(
            ins[t].at[pl.ds(r0, _CHUNK_ROWS)], ring.at[i % _NS], rsem.at[i % _NS])

    def wr(i):
        t, r0 = _CHUNKS[i]
        return pltpu.make_async_copy(
            ring.at[i % _NS], outs[t].at[pl.ds(r0, _CHUNK_ROWS)], wsem.at[i % _NS])

    def vrd(k):
        return pltpu.make_async_copy(
            vx.at[pl.ds(k * _VCHUNK, _VCHUNK)], vring.at[k % 2], vrsem.at[k % 2])

    def vwr(k):
        return pltpu.make_async_copy(
            vring.at[k % 2], ov.at[pl.ds(k * _VCHUNK, _VCHUNK)], vwsem.at[k % 2])

    vrd(0).start()
    vrd(1).start()
    for i in range(n + _LAG):
        if i < n:
            if i >= _NS:
                wr(i - _NS).wait()
            rd(i).start()
        j = i - _LAG
        if 0 <= j < n:
            rd(j).wait()
            wr(j).start()
        k = _VSTEPS.get(i)
        if k is not None:
            if k >= 2:
                vwr(k - 2).wait()
            vrd(k).wait()
            vwr(k).start()
            if k + 2 < _VN:
                vrd(k + 2).start()
    for j in range(n - _NS, n):
        wr(j).wait()
    vwr(_VN - 2).wait()
    vwr(_VN - 1).wait()


def kernel(semantic, distance, vertex, embedding, direction):
    wide = [semantic.reshape(_WIDE[0]), distance.reshape(_WIDE[1]),
            embedding.reshape(_WIDE[2]), direction.reshape(_WIDE[3])]
    vx = vertex.transpose(2, 3, 0, 1).reshape(_VSHAPE)
    outs = pl.pallas_call(
        _stream_body,
        in_specs=[pl.BlockSpec(memory_space=pl.ANY)] * 5,
        out_specs=[pl.BlockSpec(memory_space=pl.ANY)] * 5,
        out_shape=[jax.ShapeDtypeStruct(f.shape, f.dtype)
                   for f in (*wide, vx)],
        scratch_shapes=[
            pltpu.VMEM((_NS, _CHUNK_ROWS, 400), jnp.float32),
            pltpu.VMEM((2, _VCHUNK) + _VSHAPE[1:], jnp.float32),
            pltpu.SemaphoreType.DMA((_NS,)),
            pltpu.SemaphoreType.DMA((_NS,)),
            pltpu.SemaphoreType.DMA((2,)),
            pltpu.SemaphoreType.DMA((2,)),
        ],
        compiler_params=pltpu.CompilerParams(vmem_limit_bytes=60 * 1024 * 1024),
    )(*wide, vx)
    ver = outs[4].reshape(25, 50, 32, 65).transpose(2, 3, 0, 1)
    return (outs[0].reshape(semantic.shape), outs[1].reshape(distance.shape),
            ver, outs[2].reshape(embedding.shape),
            outs[3].reshape(direction.shape))
